# Initial kernel scaffold; baseline (speedup 1.0000x reference)
import jax
import jax.numpy as jnp
from jax.experimental import pallas as pl


def kernel(x, edge_index, edge_attr, batch, B_total, Wl1, bl1, Wr1, Wl2, bl2, Wr2, Wm1, bm1, Wm2, bm2):
    raise NotImplementedError("write your pallas kernel here")


# speedup vs baseline: 5.1053x; 5.1053x over previous
import functools

import jax
import jax.numpy as jnp
from jax import lax
from jax.experimental import pallas as pl
from jax.experimental.pallas import tpu as pltpu
from jax.experimental.pallas import tpu_sc as plsc

_N = 10000
_H = 64
_G = 16
_SUB = 128
_NC = 2
_NS = 16
_NW = _NC * _NS
_NPAD = 10240
_NSH = _NPAD // _NS


def _seg_inner(rpw, with_cnt, p_hbm, src_hbm, dst_hbm, z64_hbm,
               seg_out, src_idx, dst_idx, gbuf, acc, sem,
               zc_hbm=None, ones_hbm=None, cnt_out=None, ones_v=None,
               cnt_acc=None):
    c = lax.axis_index("c")
    s = lax.axis_index("s")
    wid = c * _NS + s
    base = wid * rpw
    pltpu.sync_copy(src_hbm.at[pl.ds(base, rpw)], src_idx)
    pltpu.sync_copy(dst_hbm.at[pl.ds(base, rpw)], dst_idx)
    pltpu.sync_copy(z64_hbm, acc.at[pl.ds(s * _NSH, _NSH)])
    if with_cnt:
        pltpu.sync_copy(zc_hbm, cnt_acc.at[pl.ds(s * _NSH, _NSH)])
        pltpu.sync_copy(ones_hbm, ones_v)
    plsc.subcore_barrier()

    def step(k, carry):
        pltpu.async_copy(p_hbm.at[src_idx.at[k]], gbuf, sem).wait()
        pltpu.sync_copy(gbuf, acc.at[dst_idx.at[k]], add=True)
        if with_cnt:
            pltpu.sync_copy(ones_v, cnt_acc.at[dst_idx.at[k]], add=True)
        return carry

    lax.fori_loop(0, rpw, step, 0)
    plsc.subcore_barrier()
    sl = pl.ds(s * _NSH, _NSH)
    pltpu.sync_copy(acc.at[sl], seg_out.at[c, sl])
    if with_cnt:
        pltpu.sync_copy(cnt_acc.at[sl], cnt_out.at[c, sl])


@functools.lru_cache(maxsize=None)
def _make_seg(rpw, with_cnt):
    mesh = plsc.VectorSubcoreMesh(core_axis_name="c", subcore_axis_name="s")
    out_type = [jax.ShapeDtypeStruct((_NC, _NPAD, _H), jnp.float32)]
    scratch = [
        pltpu.VMEM((rpw, _SUB), jnp.int32),
        pltpu.VMEM((rpw, _SUB), jnp.int32),
        pltpu.VMEM((_SUB, _H), jnp.float32),
        pltpu.VMEM_SHARED((_NPAD, _H), jnp.float32),
        pltpu.SemaphoreType.DMA,
    ]
    if with_cnt:
        out_type.append(jax.ShapeDtypeStruct((_NC, _NPAD, 8), jnp.float32))

        @functools.partial(pl.kernel, out_type=out_type, mesh=mesh,
                           compiler_params=pltpu.CompilerParams(
                               use_tc_tiling_on_sc=False),
                           scratch_types=scratch + [
                               pltpu.VMEM((_SUB, 8), jnp.float32),
                               pltpu.VMEM_SHARED((_NPAD, 8), jnp.float32),
                           ])
        def seg_k(p_hbm, src_hbm, dst_hbm, z64_hbm, zc_hbm, ones_hbm,
                  seg_out, cnt_out, src_idx, dst_idx, gbuf, acc, sem,
                  ones_v, cnt_acc):
            _seg_inner(rpw, True, p_hbm, src_hbm, dst_hbm, z64_hbm,
                       seg_out, src_idx, dst_idx, gbuf, acc, sem,
                       zc_hbm=zc_hbm, ones_hbm=ones_hbm, cnt_out=cnt_out,
                       ones_v=ones_v, cnt_acc=cnt_acc)
    else:
        @functools.partial(pl.kernel, out_type=out_type, mesh=mesh,
                           compiler_params=pltpu.CompilerParams(
                               use_tc_tiling_on_sc=False),
                           scratch_types=scratch)
        def seg_k(p_hbm, src_hbm, dst_hbm, z64_hbm, seg_out,
                  src_idx, dst_idx, gbuf, acc, sem):
            _seg_inner(rpw, False, p_hbm, src_hbm, dst_hbm, z64_hbm,
                       seg_out, src_idx, dst_idx, gbuf, acc, sem)

    return seg_k


_DNUM = (((1,), (1,)), ((), ()))


def _tc_a_body(x_ref, wl_ref, wr_ref, p_out, r_out):
    xv = x_ref[...]
    p_out[...] = lax.dot_general(xv, wl_ref[...], _DNUM,
                                 preferred_element_type=jnp.float32)
    r_out[...] = lax.dot_general(xv, wr_ref[...], _DNUM,
                                 preferred_element_type=jnp.float32)


def _tc_b_body(segp_ref, cntp_ref, r1_ref, bl1_ref, wl2_ref, wr2_ref,
               bl2_ref, p2_out, r2_out):
    seg = segp_ref[0] + segp_ref[1]
    cnt = cntp_ref[0, :, 0:1] + cntp_ref[1, :, 0:1]
    mean = seg / jnp.maximum(cnt, 1.0)
    h = jnp.maximum(mean + bl1_ref[...] + r1_ref[...], 0.0)
    p2_out[...] = lax.dot_general(h, wl2_ref[...], _DNUM,
                                  preferred_element_type=jnp.float32)
    r2_out[...] = lax.dot_general(h, wr2_ref[...], _DNUM,
                                  preferred_element_type=jnp.float32) + bl2_ref[...]


def _tc_c_body(segp_ref, cntp_ref, r2_ref, wm1_ref, bm1_ref, wm2_ref,
               bm2_ref, batch_ref, bt_ref, out_ref):
    seg = segp_ref[0] + segp_ref[1]
    cnt = cntp_ref[0, :, 0:1] + cntp_ref[1, :, 0:1]
    h = seg / jnp.maximum(cnt, 1.0) + r2_ref[...]
    m = jnp.maximum(lax.dot_general(h, wm1_ref[...], _DNUM,
                                    preferred_element_type=jnp.float32)
                    + bm1_ref[...], 0.0)
    z = jnp.sum(m * wm2_ref[...], axis=1, keepdims=True) + bm2_ref[...]
    pi = jax.nn.sigmoid(z)
    b = batch_ref[...]
    gid = lax.broadcasted_iota(jnp.int32, (1, _G), 1)
    onehot = (b == gid).astype(jnp.float32)
    total = jnp.sum(onehot * pi, axis=0, keepdims=True)
    ratio = jnp.minimum(bt_ref[...] / (total + 1e-12), 1.0)
    rnode = jnp.sum(onehot * ratio, axis=1, keepdims=True)
    out_ref[...] = pi * rnode


def _sds(*shape):
    return jax.ShapeDtypeStruct(shape, jnp.float32)


@jax.jit
def _impl(x, edge_index, batch, B_total,
          Wl1, bl1, Wr1, Wl2, bl2, Wr2, Wm1, bm1, Wm2, bm2):
    n, f_in = x.shape
    e = edge_index.shape[1]
    rpw = -(-e // (_NW * _SUB))
    rpw = -(-rpw // 8) * 8
    rt = rpw * _NW
    epad = rt * _SUB
    src_p = jnp.concatenate(
        [edge_index[0], jnp.zeros((epad - e,), jnp.int32)]).reshape(rt, _SUB)
    dst_p = jnp.concatenate(
        [edge_index[1], jnp.full((epad - e,), _N, jnp.int32)]).reshape(rt, _SUB)
    z64 = jnp.zeros((_NSH, _H), jnp.float32)
    zc = jnp.zeros((_NSH, 8), jnp.float32)
    ones8 = jnp.ones((_SUB, 8), jnp.float32)

    p1, r1 = pl.pallas_call(
        _tc_a_body,
        out_shape=[_sds(n, _H), _sds(n, _H)],
    )(x, Wl1, Wr1)

    seg1p, cntp = _make_seg(rpw, True)(p1, src_p, dst_p, z64, zc, ones8)
    seg1p = seg1p[:, :n, :]
    cntp = cntp[:, :n, :]

    p2, r2 = pl.pallas_call(
        _tc_b_body,
        out_shape=[_sds(n, _H), _sds(n, _H)],
    )(seg1p, cntp, r1, bl1.reshape(1, -1), Wl2, Wr2, bl2.reshape(1, -1))

    (seg2p,) = _make_seg(rpw, False)(p2, src_p, dst_p, z64)
    seg2p = seg2p[:, :n, :]

    out = pl.pallas_call(
        _tc_c_body,
        out_shape=_sds(n, 1),
    )(seg2p, cntp, r2, Wm1, bm1.reshape(1, -1), Wm2, bm2.reshape(1, -1),
      batch.reshape(-1, 1), B_total.reshape(1, -1))
    return out[:, 0]


def kernel(x, edge_index, edge_attr, batch, B_total,
           Wl1, bl1, Wr1, Wl2, bl2, Wr2, Wm1, bm1, Wm2, bm2):
    del edge_attr
    return _impl(x, edge_index, batch, B_total,
                 Wl1, bl1, Wr1, Wl2, bl2, Wr2, Wm1, bm1, Wm2, bm2)


# speedup vs baseline: 5.9878x; 1.1728x over previous
import functools

import jax
import jax.numpy as jnp
from jax import lax
from jax.experimental import pallas as pl
from jax.experimental.pallas import tpu as pltpu
from jax.experimental.pallas import tpu_sc as plsc

_N = 10000
_H = 64
_G = 16
_SUB = 128
_NC = 2
_NS = 16
_NW = _NC * _NS
_NPAD = 10240
_NSH = _NPAD // _NS


def _seg_inner(rpw, with_cnt, p_hbm, src_hbm, dst_hbm, z64_hbm,
               seg_out, src_idx, dst_idx, gbuf0, gbuf1, acc, sem0, sem1,
               zc_hbm=None, ones_hbm=None, cnt_out=None, ones_v=None,
               cnt_acc=None):
    c = lax.axis_index("c")
    s = lax.axis_index("s")
    wid = c * _NS + s
    base = wid * rpw
    pltpu.sync_copy(src_hbm.at[pl.ds(base, rpw)], src_idx)
    pltpu.sync_copy(dst_hbm.at[pl.ds(base, rpw)], dst_idx)
    pltpu.sync_copy(z64_hbm, acc.at[pl.ds(s * _NSH, _NSH)])
    if with_cnt:
        pltpu.sync_copy(zc_hbm, cnt_acc.at[pl.ds(s * _NSH, _NSH)])
        pltpu.sync_copy(ones_hbm, ones_v)
    plsc.subcore_barrier()

    dummy = p_hbm.at[pl.ds(0, _SUB)]

    def halfstep(k, gbuf, sem, start_next):
        pltpu.make_async_copy(dummy, gbuf, sem).wait()
        pltpu.sync_copy(gbuf, acc.at[dst_idx.at[k]], add=True)
        if with_cnt:
            pltpu.sync_copy(ones_v, cnt_acc.at[dst_idx.at[k]], add=True)
        if start_next:
            pltpu.async_copy(p_hbm.at[src_idx.at[k + 2]], gbuf, sem)

    pairs = rpw // 2
    pltpu.async_copy(p_hbm.at[src_idx.at[0]], gbuf0, sem0)
    pltpu.async_copy(p_hbm.at[src_idx.at[1]], gbuf1, sem1)

    def pair(j, carry):
        halfstep(2 * j, gbuf0, sem0, True)
        halfstep(2 * j + 1, gbuf1, sem1, True)
        return carry

    lax.fori_loop(0, pairs - 1, pair, 0)
    halfstep(rpw - 2, gbuf0, sem0, False)
    halfstep(rpw - 1, gbuf1, sem1, False)

    plsc.subcore_barrier()
    sl = pl.ds(s * _NSH, _NSH)
    pltpu.sync_copy(acc.at[sl], seg_out.at[c, sl])
    if with_cnt:
        pltpu.sync_copy(cnt_acc.at[sl], cnt_out.at[c, sl])


@functools.lru_cache(maxsize=None)
def _make_seg(rpw, with_cnt):
    mesh = plsc.VectorSubcoreMesh(core_axis_name="c", subcore_axis_name="s")
    out_type = [jax.ShapeDtypeStruct((_NC, _NPAD, _H), jnp.float32)]
    scratch = [
        pltpu.VMEM((rpw, _SUB), jnp.int32),
        pltpu.VMEM((rpw, _SUB), jnp.int32),
        pltpu.VMEM((_SUB, _H), jnp.float32),
        pltpu.VMEM((_SUB, _H), jnp.float32),
        pltpu.VMEM_SHARED((_NPAD, _H), jnp.float32),
        pltpu.SemaphoreType.DMA,
        pltpu.SemaphoreType.DMA,
    ]
    if with_cnt:
        out_type.append(jax.ShapeDtypeStruct((_NC, _NPAD, 8), jnp.float32))

        @functools.partial(pl.kernel, out_type=out_type, mesh=mesh,
                           compiler_params=pltpu.CompilerParams(
                               use_tc_tiling_on_sc=False),
                           scratch_types=scratch + [
                               pltpu.VMEM((_SUB, 8), jnp.float32),
                               pltpu.VMEM_SHARED((_NPAD, 8), jnp.float32),
                           ])
        def seg_k(p_hbm, src_hbm, dst_hbm, z64_hbm, zc_hbm, ones_hbm,
                  seg_out, cnt_out, src_idx, dst_idx, gbuf0, gbuf1, acc,
                  sem0, sem1, ones_v, cnt_acc):
            _seg_inner(rpw, True, p_hbm, src_hbm, dst_hbm, z64_hbm,
                       seg_out, src_idx, dst_idx, gbuf0, gbuf1, acc,
                       sem0, sem1,
                       zc_hbm=zc_hbm, ones_hbm=ones_hbm, cnt_out=cnt_out,
                       ones_v=ones_v, cnt_acc=cnt_acc)
    else:
        @functools.partial(pl.kernel, out_type=out_type, mesh=mesh,
                           compiler_params=pltpu.CompilerParams(
                               use_tc_tiling_on_sc=False),
                           scratch_types=scratch)
        def seg_k(p_hbm, src_hbm, dst_hbm, z64_hbm, seg_out,
                  src_idx, dst_idx, gbuf0, gbuf1, acc, sem0, sem1):
            _seg_inner(rpw, False, p_hbm, src_hbm, dst_hbm, z64_hbm,
                       seg_out, src_idx, dst_idx, gbuf0, gbuf1, acc,
                       sem0, sem1)

    return seg_k


_DNUM = (((1,), (1,)), ((), ()))


def _tc_a_body(x_ref, wl_ref, wr_ref, p_out, r_out):
    xv = x_ref[...]
    p_out[...] = lax.dot_general(xv, wl_ref[...], _DNUM,
                                 preferred_element_type=jnp.float32)
    r_out[...] = lax.dot_general(xv, wr_ref[...], _DNUM,
                                 preferred_element_type=jnp.float32)


def _tc_b_body(segp_ref, cntp_ref, r1_ref, bl1_ref, wl2_ref, wr2_ref,
               bl2_ref, p2_out, r2_out):
    seg = segp_ref[0] + segp_ref[1]
    cnt = cntp_ref[0, :, 0:1] + cntp_ref[1, :, 0:1]
    mean = seg / jnp.maximum(cnt, 1.0)
    h = jnp.maximum(mean + bl1_ref[...] + r1_ref[...], 0.0)
    p2_out[...] = lax.dot_general(h, wl2_ref[...], _DNUM,
                                  preferred_element_type=jnp.float32)
    r2_out[...] = lax.dot_general(h, wr2_ref[...], _DNUM,
                                  preferred_element_type=jnp.float32) + bl2_ref[...]


def _tc_c_body(segp_ref, cntp_ref, r2_ref, wm1_ref, bm1_ref, wm2_ref,
               bm2_ref, batch_ref, bt_ref, out_ref):
    seg = segp_ref[0] + segp_ref[1]
    cnt = cntp_ref[0, :, 0:1] + cntp_ref[1, :, 0:1]
    h = seg / jnp.maximum(cnt, 1.0) + r2_ref[...]
    m = jnp.maximum(lax.dot_general(h, wm1_ref[...], _DNUM,
                                    preferred_element_type=jnp.float32)
                    + bm1_ref[...], 0.0)
    z = jnp.sum(m * wm2_ref[...], axis=1, keepdims=True) + bm2_ref[...]
    pi = jax.nn.sigmoid(z)
    b = batch_ref[...]
    gid = lax.broadcasted_iota(jnp.int32, (1, _G), 1)
    onehot = (b == gid).astype(jnp.float32)
    total = jnp.sum(onehot * pi, axis=0, keepdims=True)
    ratio = jnp.minimum(bt_ref[...] / (total + 1e-12), 1.0)
    rnode = jnp.sum(onehot * ratio, axis=1, keepdims=True)
    out_ref[...] = pi * rnode


def _sds(*shape):
    return jax.ShapeDtypeStruct(shape, jnp.float32)


@jax.jit
def _impl(x, edge_index, batch, B_total,
          Wl1, bl1, Wr1, Wl2, bl2, Wr2, Wm1, bm1, Wm2, bm2):
    n, f_in = x.shape
    e = edge_index.shape[1]
    rpw = -(-e // (_NW * _SUB))
    rpw = -(-rpw // 8) * 8
    rt = rpw * _NW
    epad = rt * _SUB
    src_p = jnp.concatenate(
        [edge_index[0], jnp.zeros((epad - e,), jnp.int32)]).reshape(rt, _SUB)
    dst_p = jnp.concatenate(
        [edge_index[1], jnp.full((epad - e,), _N, jnp.int32)]).reshape(rt, _SUB)
    z64 = jnp.zeros((_NSH, _H), jnp.float32)
    zc = jnp.zeros((_NSH, 8), jnp.float32)
    ones8 = jnp.ones((_SUB, 8), jnp.float32)

    p1, r1 = pl.pallas_call(
        _tc_a_body,
        out_shape=[_sds(n, _H), _sds(n, _H)],
    )(x, Wl1, Wr1)

    seg1p, cntp = _make_seg(rpw, True)(p1, src_p, dst_p, z64, zc, ones8)
    seg1p = seg1p[:, :n, :]
    cntp = cntp[:, :n, :]

    p2, r2 = pl.pallas_call(
        _tc_b_body,
        out_shape=[_sds(n, _H), _sds(n, _H)],
    )(seg1p, cntp, r1, bl1.reshape(1, -1), Wl2, Wr2, bl2.reshape(1, -1))

    (seg2p,) = _make_seg(rpw, False)(p2, src_p, dst_p, z64)
    seg2p = seg2p[:, :n, :]

    out = pl.pallas_call(
        _tc_c_body,
        out_shape=_sds(n, 1),
    )(seg2p, cntp, r2, Wm1, bm1.reshape(1, -1), Wm2, bm2.reshape(1, -1),
      batch.reshape(-1, 1), B_total.reshape(1, -1))
    return out[:, 0]


def kernel(x, edge_index, edge_attr, batch, B_total,
           Wl1, bl1, Wr1, Wl2, bl2, Wr2, Wm1, bm1, Wm2, bm2):
    del edge_attr
    return _impl(x, edge_index, batch, B_total,
                 Wl1, bl1, Wr1, Wl2, bl2, Wr2, Wm1, bm1, Wm2, bm2)
